# baseline (speedup 1.0000x reference)
import functools

import jax
import jax.numpy as jnp
from jax import lax
from jax.experimental import pallas as pl
from jax.experimental.pallas import tpu as pltpu
from jax.experimental.pallas import tpu_sc as plsc


def _sc_copy(pe2d):
    L, D = pe2d.shape
    info = plsc.get_sparse_core_info()
    nc = info.num_cores
    rows_per_c = L // nc

    mesh = plsc.ScalarSubcoreMesh(axis_name="c", num_cores=nc)

    @functools.partial(
        pl.kernel,
        out_type=jax.ShapeDtypeStruct((L, D), pe2d.dtype),
        mesh=mesh,
        scratch_types=[pltpu.VMEM_SHARED((rows_per_c, D), pe2d.dtype)],
    )
    def copy_kernel(pe_hbm, out_hbm, buf):
        cid = lax.axis_index("c")
        base = cid * rows_per_c
        pltpu.sync_copy(pe_hbm.at[pl.ds(base, rows_per_c)], buf)
        pltpu.sync_copy(buf, out_hbm.at[pl.ds(base, rows_per_c)])

    return copy_kernel(pe2d)


def kernel(x, pe):
    L = x.shape[1]
    pe2d = pe.reshape(pe.shape[1], pe.shape[2])[:L]
    return _sc_copy(pe2d)[None]
